# Initial kernel scaffold; baseline (speedup 1.0000x reference)
import functools

import jax
import jax.numpy as jnp
from jax import lax
from jax.experimental import pallas as pl
from jax.experimental.pallas import tpu as pltpu
from jax.experimental.pallas import tpu_sc as plsc

LANES = 16


def _lane_bcast(vec, lane):
    return jnp.take(vec, jnp.full((LANES,), lane, jnp.int32),
                    mode="promise_in_bounds")


def _make_kernel(B, L, D, V, num_workers, cb):
    rows_per_worker = B // num_workers
    n_chunks = rows_per_worker // cb
    n_idx = cb * L
    d_regs = D // LANES
    splits = []
    off = 0
    while off < n_idx:
        n = min(128, n_idx - off)
        splits.append((off, n))
        off += n

    mesh = plsc.VectorSubcoreMesh(core_axis_name="c", subcore_axis_name="s")

    @functools.partial(
        pl.kernel,
        out_type=jax.ShapeDtypeStruct((B, D), jnp.float32),
        mesh=mesh,
        scratch_types=[
            pltpu.VMEM((n_idx,), jnp.int32),
            pltpu.VMEM((cb, D), jnp.float32),
            pltpu.VMEM((cb, D), jnp.int32),
            pltpu.VMEM((n_idx, D), jnp.float32),
            pltpu.VMEM((cb, D), jnp.float32),
            pltpu.SemaphoreType.DMA,
        ],
    )
    def kern(ids_flat, w_pad, idm_pad, table, out, idx_v, w_v, im_v,
             emb_v, out_v, sem):
        nc = mesh.num_cores
        wid = lax.axis_index("s") * nc + lax.axis_index("c")
        row0 = wid * rows_per_worker

        def chunk_body(g, _):
            rbase = row0 + g * cb

            pltpu.sync_copy(ids_flat.at[pl.ds(rbase * L, n_idx)], idx_v)
            pltpu.sync_copy(w_pad.at[pl.ds(rbase, cb)], w_v)
            pltpu.sync_copy(idm_pad.at[pl.ds(rbase, cb)], im_v)

            copies = [
                pltpu.async_copy(
                    table.at[idx_v.at[pl.ds(o, n)]],
                    emb_v.at[pl.ds(o, n)], sem)
                for (o, n) in splits
            ]
            for c in copies:
                c.wait()

            def row_body(r, _):
                wv = [w_v[r, pl.ds(k * LANES, LANES)] for k in range(d_regs)]
                iv = [im_v[r, pl.ds(k * LANES, LANES)] for k in range(d_regs)]
                mw = [jnp.where(iv[k] == 0, jnp.float32(-1e9), wv[k])
                      for k in range(d_regs)]
                mx = mw[0]
                for k in range(1, d_regs):
                    mx = jnp.maximum(mx, mw[k])
                m = lax.reduce_max(mx, (0,))
                ev = [jnp.exp(mw[k] - m) for k in range(d_regs)]
                zs = ev[0]
                for k in range(1, d_regs):
                    zs = zs + ev[k]
                z = lax.reduce_sum(zs, (0,))
                acc = [jnp.zeros((LANES,), jnp.float32)
                       for _ in range(d_regs)]
                for l in range(L):
                    s = _lane_bcast(ev[l // LANES], l % LANES)
                    ebase = r * L + l
                    for k in range(d_regs):
                        acc[k] = acc[k] + s * emb_v[ebase,
                                                    pl.ds(k * LANES, LANES)]
                inv_z = jnp.float32(1.0) / z
                for k in range(d_regs):
                    out_v[r, pl.ds(k * LANES, LANES)] = acc[k] * inv_z
                return 0

            lax.fori_loop(0, cb, row_body, 0, unroll=False)
            pltpu.sync_copy(out_v, out.at[pl.ds(rbase, cb)])
            return 0

        lax.fori_loop(0, n_chunks, chunk_body, 0, unroll=False)

    return kern


def kernel(ids, weights, table):
    B, L = ids.shape
    V, D = table.shape
    ids = ids.astype(jnp.int32)
    ids_flat = ids.reshape(B * L)
    w_pad = jnp.pad(weights, ((0, 0), (0, D - L)),
                    constant_values=-jnp.inf)
    idm_pad = jnp.pad(ids, ((0, 0), (0, D - L)), constant_values=1)
    info = plsc.get_sparse_core_info()
    num_workers = info.num_cores * info.num_subcores
    kern = _make_kernel(B, L, D, V, num_workers, cb=8)
    return kern(ids_flat, w_pad, idm_pad, table)


# speedup vs baseline: 2.1873x; 2.1873x over previous
import functools

import jax
import jax.numpy as jnp
from jax import lax
from jax.experimental import pallas as pl
from jax.experimental.pallas import tpu as pltpu
from jax.experimental.pallas import tpu_sc as plsc

LANES = 16


def _lane_bcast(vec, lane):
    idx = jnp.full((LANES,), lane, jnp.int32)
    return vec.at[idx].get(mode="promise_in_bounds")


def _lane_reduce(vec, op):
    idx = lax.iota(jnp.int32, LANES)
    for sh in (8, 4, 2, 1):
        shuf = vec.at[idx ^ sh].get(mode="promise_in_bounds")
        vec = op(vec, shuf)
    return vec


def _make_kernel(B, L, D, V, num_workers, cb):
    rows_per_worker = B // num_workers
    n_chunks = rows_per_worker // cb
    n_idx = cb * L
    d_regs = D // LANES
    splits = []
    off = 0
    while off < n_idx:
        n = min(128, n_idx - off)
        splits.append((off, n))
        off += n

    mesh = plsc.VectorSubcoreMesh(core_axis_name="c", subcore_axis_name="s")

    @functools.partial(
        pl.kernel,
        out_type=jax.ShapeDtypeStruct((B, D), jnp.float32),
        mesh=mesh,
        scratch_types=[
            pltpu.VMEM((n_idx,), jnp.int32),
            pltpu.VMEM((cb, D), jnp.float32),
            pltpu.VMEM((cb, D), jnp.int32),
            pltpu.VMEM((n_idx, D), jnp.float32),
            pltpu.VMEM((cb, D), jnp.float32),
            pltpu.SemaphoreType.DMA,
        ],
        compiler_params=pltpu.CompilerParams(use_tc_tiling_on_sc=False),
    )
    def kern(ids_flat, w_pad, idm_pad, table, out, idx_v, w_v, im_v,
             emb_v, out_v, sem):
        nc = mesh.num_cores
        wid = lax.axis_index("s") * nc + lax.axis_index("c")
        row0 = wid * rows_per_worker

        def chunk_body(g, _):
            rbase = row0 + g * cb

            pltpu.sync_copy(ids_flat.at[pl.ds(rbase * L, n_idx)], idx_v)
            pltpu.sync_copy(w_pad.at[pl.ds(rbase, cb)], w_v)
            pltpu.sync_copy(idm_pad.at[pl.ds(rbase, cb)], im_v)

            copies = [
                pltpu.async_copy(
                    table.at[idx_v.at[pl.ds(o, n)]],
                    emb_v.at[pl.ds(o, n)], sem)
                for (o, n) in splits
            ]
            for c in copies:
                c.wait()

            def row_body(r, _):
                wv = [w_v[r, pl.ds(k * LANES, LANES)] for k in range(d_regs)]
                iv = [im_v[r, pl.ds(k * LANES, LANES)] for k in range(d_regs)]
                mw = [jnp.where(iv[k] == 0, jnp.float32(-1e9), wv[k])
                      for k in range(d_regs)]
                mx = mw[0]
                for k in range(1, d_regs):
                    mx = jnp.maximum(mx, mw[k])
                m = _lane_reduce(mx, jnp.maximum)
                ev = [jnp.exp(mw[k] - m) for k in range(d_regs)]
                zs = ev[0]
                for k in range(1, d_regs):
                    zs = zs + ev[k]
                z = _lane_reduce(zs, jnp.add)
                acc = [jnp.zeros((LANES,), jnp.float32)
                       for _ in range(d_regs)]
                for l in range(L):
                    s = _lane_bcast(ev[l // LANES], l % LANES)
                    ebase = r * L + l
                    for k in range(d_regs):
                        acc[k] = acc[k] + s * emb_v[ebase,
                                                    pl.ds(k * LANES, LANES)]
                inv_z = jnp.float32(1.0) / z
                for k in range(d_regs):
                    out_v[r, pl.ds(k * LANES, LANES)] = acc[k] * inv_z
                return 0

            lax.fori_loop(0, cb, row_body, 0, unroll=False)
            pltpu.sync_copy(out_v, out.at[pl.ds(rbase, cb)])
            return 0

        lax.fori_loop(0, n_chunks, chunk_body, 0, unroll=False)

    return kern


def kernel(ids, weights, table):
    B, L = ids.shape
    V, D = table.shape
    ids = ids.astype(jnp.int32)
    ids_flat = ids.reshape(B * L)
    w_pad = jnp.pad(weights, ((0, 0), (0, D - L)),
                    constant_values=-jnp.inf)
    idm_pad = jnp.pad(ids, ((0, 0), (0, D - L)), constant_values=1)
    info = plsc.get_sparse_core_info()
    num_workers = info.num_cores * info.num_subcores
    kern = _make_kernel(B, L, D, V, num_workers, cb=8)
    return kern(ids_flat, w_pad, idm_pad, table)
